# baseline (speedup 1.0000x reference)
import jax
import jax.numpy as jnp
from jax import lax
from jax.experimental import pallas as pl
from jax.experimental.pallas import tpu as pltpu
from jax.experimental.pallas import tpu_sc as plsc

N_DENSE = 13
N_SPARSE = 26
VOCAB = 100000
EMBED = 16
B = 16384
H1, H2, OUT = 128, 64, 1
D_IN = EMBED + N_DENSE

NC, NS = 2, 16
NW = NC * NS
NHALF = 2
HB = B // NHALF
LANES = 16
VCUT = 50048
W1S = 49920
WIN = 50048
TAIL0 = VOCAB - 128
PBUF = WIN + 32


def _sc_pool_body(planes_hbm, tail_hbm, idx_hbm, inter_t_hbm,
                  buf_lo, buf_hi, idx_v, s_v, q_v, sem_lo, sem_hi):
    cid = lax.axis_index("c")
    sid = lax.axis_index("s")
    wid = sid * NC + cid
    e = wid // NHALF
    h = wid % NHALF

    cut_vec = jnp.full((LANES,), 0, jnp.int32) + VCUT
    off_vec = jnp.full((LANES,), 0, jnp.int32) + W1S
    zero = jnp.zeros((LANES,), jnp.float32)

    def zbody(i, carry):
        s_v[pl.ds(i * LANES, LANES)] = zero
        q_v[pl.ds(i * LANES, LANES)] = zero
        return carry
    lax.fori_loop(0, HB // LANES, zbody, 0, unroll=8)

    def start_lo(f):
        row = f * EMBED + e
        return pltpu.async_copy(planes_hbm.at[row, pl.ds(0, WIN)],
                                buf_lo.at[pl.ds(0, WIN)], sem_lo)

    def start_hi(f):
        row = f * EMBED + e
        d1 = pltpu.async_copy(planes_hbm.at[row, pl.ds(W1S, WIN)],
                              buf_hi.at[pl.ds(0, WIN)], sem_hi)
        d2 = pltpu.async_copy(tail_hbm.at[pl.ds(row * 128, 128)],
                              buf_hi.at[pl.ds(PBUF - 128, 128)], sem_hi)
        return d1, d2

    def pass_lo(carry):
        def gbody(i, c2):
            sl = pl.ds(i * LANES, LANES)
            ids = idx_v[sl]
            m = ids < cut_vec
            v = plsc.load_gather(buf_lo, [ids], mask=m)
            v = jnp.where(m, v, 0.0)
            plsc.addupdate(s_v.at[sl], v)
            plsc.addupdate(q_v.at[sl], v * v)
            return c2
        lax.fori_loop(0, HB // LANES, gbody, carry, unroll=4)

    def pass_hi(carry):
        def gbody(i, c2):
            sl = pl.ds(i * LANES, LANES)
            ids = idx_v[sl]
            m = ids >= cut_vec
            loc = ids - off_vec
            v = plsc.load_gather(buf_hi, [loc], mask=m)
            v = jnp.where(m, v, 0.0)
            plsc.addupdate(s_v.at[sl], v)
            plsc.addupdate(q_v.at[sl], v * v)
            return c2
        lax.fori_loop(0, HB // LANES, gbody, carry, unroll=4)

    d_lo = start_lo(0)
    d_hi = start_hi(0)
    for f in range(N_SPARSE):
        pltpu.sync_copy(idx_hbm.at[f, h], idx_v)
        d_lo.wait()
        pass_lo(0)
        if f + 1 < N_SPARSE:
            d_lo = start_lo(f + 1)
        d_hi[0].wait()
        d_hi[1].wait()
        pass_hi(0)
        if f + 1 < N_SPARSE:
            d_hi = start_hi(f + 1)

    def finbody(i, carry):
        sl = pl.ds(i * LANES, LANES)
        s = s_v[sl]
        q = q_v[sl]
        s_v[sl] = 0.5 * (s * s - q)
        return carry
    lax.fori_loop(0, HB // LANES, finbody, 0, unroll=8)

    pltpu.sync_copy(s_v, inter_t_hbm.at[e, pl.ds(h * HB, HB)])


@jax.jit
def _sc_pool(planes, tail, idx_t):
    mesh = plsc.VectorSubcoreMesh(core_axis_name="c", subcore_axis_name="s")
    return pl.kernel(
        _sc_pool_body,
        out_type=jax.ShapeDtypeStruct((EMBED, B), jnp.float32),
        mesh=mesh,
        compiler_params=pltpu.CompilerParams(needs_layout_passes=False),
        scratch_types=[
            pltpu.VMEM((PBUF,), jnp.float32),
            pltpu.VMEM((PBUF,), jnp.float32),
            pltpu.VMEM((HB,), jnp.int32),
            pltpu.VMEM((HB,), jnp.float32),
            pltpu.VMEM((HB,), jnp.float32),
            pltpu.SemaphoreType.DMA,
            pltpu.SemaphoreType.DMA,
        ],
    )(planes, tail, idx_t)


BS = 2048


def _mlp_body(it_ref, d_ref,
              g16_ref, be16_ref, mm16_ref, iv16_ref,
              g13_ref, be13_ref, mm13_ref, iv13_ref,
              w1a_ref, w1b_ref, b1_ref, w2_ref, b2_ref, w3_ref, b3_ref,
              o_ref):
    inter = it_ref[...]
    inter = (inter - mm16_ref[...]) * iv16_ref[...] * g16_ref[...] + be16_ref[...]
    xd = d_ref[...]
    xd = (xd - mm13_ref[...]) * iv13_ref[...] * g13_ref[...] + be13_ref[...]
    h = lax.dot_general(inter, w1a_ref[...], (((0,), (0,)), ((), ())),
                        preferred_element_type=jnp.float32)
    h = h + jnp.dot(xd, w1b_ref[...], preferred_element_type=jnp.float32)
    h = jnp.maximum(h + b1_ref[...], 0.0)
    h = jnp.dot(h, w2_ref[...], preferred_element_type=jnp.float32) + b2_ref[...]
    h = jnp.maximum(h, 0.0)
    o = jnp.dot(h, w3_ref[...], preferred_element_type=jnp.float32) + b3_ref[...]
    o_ref[...] = jax.nn.sigmoid(o)


@jax.jit
def _mlp(inter_t, dense, gamma, beta, mm, mv, W1, b1, W2, b2, W3, b3):
    inv = lax.rsqrt(mv + 1e-3)
    col = lambda a: a[:EMBED].reshape(EMBED, 1)
    rowp = lambda a: a[EMBED:].reshape(1, N_DENSE)
    full = lambda shape: pl.BlockSpec(shape, lambda i: (0,) * len(shape))
    return pl.pallas_call(
        _mlp_body,
        grid=(B // BS,),
        in_specs=[
            pl.BlockSpec((EMBED, BS), lambda i: (0, i)),
            pl.BlockSpec((BS, N_DENSE), lambda i: (i, 0)),
            full((EMBED, 1)), full((EMBED, 1)), full((EMBED, 1)), full((EMBED, 1)),
            full((1, N_DENSE)), full((1, N_DENSE)), full((1, N_DENSE)), full((1, N_DENSE)),
            full((EMBED, H1)), full((N_DENSE, H1)), full((1, H1)),
            full((H1, H2)), full((1, H2)),
            full((H2, OUT)), full((1, OUT)),
        ],
        out_specs=pl.BlockSpec((BS, OUT), lambda i: (i, 0)),
        out_shape=jax.ShapeDtypeStruct((B, OUT), jnp.float32),
    )(inter_t, dense,
      col(gamma), col(beta), col(mm), col(inv),
      rowp(gamma), rowp(beta), rowp(mm), rowp(inv),
      W1[:EMBED], W1[EMBED:], b1.reshape(1, H1),
      W2, b2.reshape(1, H2), W3, b3.reshape(1, OUT))


def kernel(inputs, embed_tables, gamma, beta, moving_mean, moving_var,
           W1, b1, W2, b2, W3, b3):
    planes = embed_tables.transpose(0, 2, 1).reshape(N_SPARSE * EMBED, VOCAB)
    tail = planes[:, TAIL0:].reshape(-1)
    sparse_idx = inputs[:, N_DENSE:].astype(jnp.int32)
    idx_t = sparse_idx.T.reshape(N_SPARSE, NHALF, HB)

    inter_t = _sc_pool(planes, tail, idx_t)
    dense = inputs[:, :N_DENSE]
    return _mlp(inter_t, dense, gamma, beta, moving_mean, moving_var,
                W1, b1, W2, b2, W3, b3)


# speedup vs baseline: 1.0160x; 1.0160x over previous
import jax
import jax.numpy as jnp
from jax import lax
from jax.experimental import pallas as pl
from jax.experimental.pallas import tpu as pltpu
from jax.experimental.pallas import tpu_sc as plsc

N_DENSE = 13
N_SPARSE = 26
VOCAB = 100000
EMBED = 16
B = 16384
H1, H2, OUT = 128, 64, 1
D_IN = EMBED + N_DENSE

NC, NS = 2, 16
NW = NC * NS
NHALF = 2
HB = B // NHALF
LANES = 16
VCUT = 50048
W1S = 49920
WIN = 50048
TAIL0 = VOCAB - 128
PBUF = WIN + 32


def _sc_pool_body(planes_hbm, tail_hbm, idx_hbm, inter_t_hbm,
                  buf_lo, buf_hi, idx_v, s_v, q_v, sem_lo, sem_hi):
    cid = lax.axis_index("c")
    sid = lax.axis_index("s")
    wid = sid * NC + cid
    e = wid // NHALF
    h = wid % NHALF

    cut_vec = jnp.full((LANES,), 0, jnp.int32) + VCUT
    off_vec = jnp.full((LANES,), 0, jnp.int32) + W1S
    zero = jnp.zeros((LANES,), jnp.float32)

    def zbody(i, carry):
        s_v[pl.ds(i * LANES, LANES)] = zero
        q_v[pl.ds(i * LANES, LANES)] = zero
        return carry
    lax.fori_loop(0, HB // LANES, zbody, 0)

    def start_lo(f):
        row = f * EMBED + e
        return pltpu.async_copy(planes_hbm.at[row, pl.ds(0, WIN)],
                                buf_lo.at[pl.ds(0, WIN)], sem_lo)

    def start_hi(f):
        row = f * EMBED + e
        d1 = pltpu.async_copy(planes_hbm.at[row, pl.ds(W1S, WIN)],
                              buf_hi.at[pl.ds(0, WIN)], sem_hi)
        d2 = pltpu.async_copy(tail_hbm.at[pl.ds(row * 128, 128)],
                              buf_hi.at[pl.ds(PBUF - 128, 128)], sem_hi)
        return d1, d2

    def pass_lo(carry):
        def gbody(i, c2):
            sl = pl.ds(i * LANES, LANES)
            ids = idx_v[sl]
            m = ids < cut_vec
            v = plsc.load_gather(buf_lo, [ids], mask=m)
            v = jnp.where(m, v, 0.0)
            plsc.addupdate(s_v.at[sl], v)
            plsc.addupdate(q_v.at[sl], v * v)
            return c2
        lax.fori_loop(0, HB // LANES, gbody, carry)

    def pass_hi(carry):
        def gbody(i, c2):
            sl = pl.ds(i * LANES, LANES)
            ids = idx_v[sl]
            m = ids >= cut_vec
            loc = ids - off_vec
            v = plsc.load_gather(buf_hi, [loc], mask=m)
            v = jnp.where(m, v, 0.0)
            plsc.addupdate(s_v.at[sl], v)
            plsc.addupdate(q_v.at[sl], v * v)
            return c2
        lax.fori_loop(0, HB // LANES, gbody, carry)

    d_lo = start_lo(0)
    d_hi = start_hi(0)
    for f in range(N_SPARSE):
        pltpu.sync_copy(idx_hbm.at[f, h], idx_v)
        d_lo.wait()
        pass_lo(0)
        if f + 1 < N_SPARSE:
            d_lo = start_lo(f + 1)
        d_hi[0].wait()
        d_hi[1].wait()
        pass_hi(0)
        if f + 1 < N_SPARSE:
            d_hi = start_hi(f + 1)

    def finbody(i, carry):
        sl = pl.ds(i * LANES, LANES)
        s = s_v[sl]
        q = q_v[sl]
        s_v[sl] = 0.5 * (s * s - q)
        return carry
    lax.fori_loop(0, HB // LANES, finbody, 0)

    pltpu.sync_copy(s_v, inter_t_hbm.at[e, pl.ds(h * HB, HB)])


@jax.jit
def _sc_pool(planes, tail, idx_t):
    mesh = plsc.VectorSubcoreMesh(core_axis_name="c", subcore_axis_name="s")
    return pl.kernel(
        _sc_pool_body,
        out_type=jax.ShapeDtypeStruct((EMBED, B), jnp.float32),
        mesh=mesh,
        compiler_params=pltpu.CompilerParams(needs_layout_passes=False),
        scratch_types=[
            pltpu.VMEM((PBUF,), jnp.float32),
            pltpu.VMEM((PBUF,), jnp.float32),
            pltpu.VMEM((HB,), jnp.int32),
            pltpu.VMEM((HB,), jnp.float32),
            pltpu.VMEM((HB,), jnp.float32),
            pltpu.SemaphoreType.DMA,
            pltpu.SemaphoreType.DMA,
        ],
    )(planes, tail, idx_t)


BS = 2048


def _mlp_body(it_ref, xt_ref,
              g16_ref, be16_ref, mm16_ref, iv16_ref,
              g13_ref, be13_ref, mm13_ref, iv13_ref,
              w1a_ref, w1b_ref, b1_ref, w2_ref, b2_ref, w3_ref, b3_ref,
              o_ref):
    inter = it_ref[...]
    inter = (inter - mm16_ref[...]) * iv16_ref[...] * g16_ref[...] + be16_ref[...]
    xd = xt_ref[...][:N_DENSE]
    xd = (xd - mm13_ref[...]) * iv13_ref[...] * g13_ref[...] + be13_ref[...]
    h = lax.dot_general(inter, w1a_ref[...], (((0,), (0,)), ((), ())),
                        preferred_element_type=jnp.float32)
    h = h + lax.dot_general(xd, w1b_ref[...], (((0,), (0,)), ((), ())),
                            preferred_element_type=jnp.float32)
    h = jnp.maximum(h + b1_ref[...], 0.0)
    h = jnp.dot(h, w2_ref[...], preferred_element_type=jnp.float32) + b2_ref[...]
    h = jnp.maximum(h, 0.0)
    o = jnp.dot(h, w3_ref[...], preferred_element_type=jnp.float32) + b3_ref[...]
    o_ref[...] = jax.nn.sigmoid(o)


@jax.jit
def _mlp(inter_t, x_t, gamma, beta, mm, mv, W1, b1, W2, b2, W3, b3):
    inv = lax.rsqrt(mv + 1e-3)
    col = lambda a: a[:EMBED].reshape(EMBED, 1)
    rowp = lambda a: a[EMBED:].reshape(N_DENSE, 1)
    full = lambda shape: pl.BlockSpec(shape, lambda i: (0,) * len(shape))
    return pl.pallas_call(
        _mlp_body,
        grid=(B // BS,),
        in_specs=[
            pl.BlockSpec((EMBED, BS), lambda i: (0, i)),
            pl.BlockSpec((N_DENSE + N_SPARSE, BS), lambda i: (0, i)),
            full((EMBED, 1)), full((EMBED, 1)), full((EMBED, 1)), full((EMBED, 1)),
            full((N_DENSE, 1)), full((N_DENSE, 1)), full((N_DENSE, 1)), full((N_DENSE, 1)),
            full((EMBED, H1)), full((N_DENSE, H1)), full((1, H1)),
            full((H1, H2)), full((1, H2)),
            full((H2, OUT)), full((1, OUT)),
        ],
        out_specs=pl.BlockSpec((BS, OUT), lambda i: (i, 0)),
        out_shape=jax.ShapeDtypeStruct((B, OUT), jnp.float32),
    )(inter_t, x_t,
      col(gamma), col(beta), col(mm), col(inv),
      rowp(gamma), rowp(beta), rowp(mm), rowp(inv),
      W1[:EMBED], W1[EMBED:], b1.reshape(1, H1),
      W2, b2.reshape(1, H2), W3, b3.reshape(1, OUT))


def kernel(inputs, embed_tables, gamma, beta, moving_mean, moving_var,
           W1, b1, W2, b2, W3, b3):
    planes = embed_tables.transpose(0, 2, 1).reshape(N_SPARSE * EMBED, VOCAB)
    tail = planes[:, TAIL0:].reshape(-1)
    sparse_idx = inputs[:, N_DENSE:].astype(jnp.int32)
    idx_t = sparse_idx.T.reshape(N_SPARSE, NHALF, HB)

    inter_t = _sc_pool(planes, tail, idx_t)
    x_t = inputs.T
    return _mlp(inter_t, x_t, gamma, beta, moving_mean, moving_var,
                W1, b1, W2, b2, W3, b3)
